# baseline (speedup 1.0000x reference)
import functools

import jax
import jax.numpy as jnp
from jax import lax
from jax.experimental import pallas as pl
from jax.experimental.pallas import tpu as pltpu
from jax.experimental.pallas import tpu_sc as plsc

N = 10000
E = 320000
CAT = 1000

NW = 32
NSUB = 16
NP = 10240
ECH = 128
NCHE = 80
VCH = 4
EPW = (NCHE + VCH) * ECH
GCH = 64
NCHG = 5
ROWS_PER_SUB = NP // NSUB
NBUF = 5


def _sc_mesh():
    return plsc.VectorSubcoreMesh(core_axis_name="c", subcore_axis_name="s")


def _sc_front(idx_u, idx_c, dst_pad, emb_user, emb_cat, zeros16, ones16):
    @functools.partial(
        pl.kernel,
        mesh=_sc_mesh(),
        compiler_params=pltpu.CompilerParams(use_tc_tiling_on_sc=False),
        out_type=[
            jax.ShapeDtypeStruct((NP, 64), jnp.float32),
            jax.ShapeDtypeStruct((NP, 32), jnp.float32),
            jax.ShapeDtypeStruct((2, NP, 16), jnp.float32),
        ],
        scratch_types=(
            [pltpu.VMEM((GCH,), jnp.int32) for _ in range(2 * NCHG)]
            + [pltpu.VMEM((GCH, 64), jnp.float32) for _ in range(NCHG)]
            + [pltpu.VMEM((GCH, 32), jnp.float32) for _ in range(NCHG)]
            + [pltpu.VMEM((ECH,), jnp.int32) for _ in range(8)]
            + [pltpu.VMEM((ECH, 16), jnp.float32),
               pltpu.VMEM_SHARED((NP, 16), jnp.float32)]
            + [pltpu.SemaphoreType.DMA for _ in range(3 + 8)]
        ),
    )
    def k(idx_u_hbm, idx_c_hbm, dst_hbm, emb_u_hbm, emb_c_hbm,
          zeros16_hbm, ones16_hbm,
          eu_hbm, ec_hbm, deg_hbm, *scr):
        iu = scr[0:NCHG]
        ic = scr[NCHG:2 * NCHG]
        bu = scr[2 * NCHG:3 * NCHG]
        bc = scr[3 * NCHG:4 * NCHG]
        didx = scr[4 * NCHG:4 * NCHG + 8]
        ones_v = scr[4 * NCHG + 8]
        deg_sh = scr[4 * NCHG + 9]
        sems = scr[4 * NCHG + 10:]
        si, sg, swb = sems[0], sems[1], sems[2]
        sd = sems[3:11]

        cid = lax.axis_index("c")
        sid = lax.axis_index("s")
        wid = cid * NSUB + sid
        gbase = wid * (NCHG * GCH)
        ebase = wid * EPW

        for t in range(NCHG):
            pltpu.async_copy(idx_u_hbm.at[pl.ds(gbase + t * GCH, GCH)],
                             iu[t], si)
            pltpu.async_copy(idx_c_hbm.at[pl.ds(gbase + t * GCH, GCH)],
                             ic[t], si)
        pltpu.async_copy(ones16_hbm, ones_v, si)

        pltpu.sync_copy(zeros16_hbm.at[pl.ds(sid * ROWS_PER_SUB, ROWS_PER_SUB)],
                        deg_sh.at[pl.ds(sid * ROWS_PER_SUB, ROWS_PER_SUB)])

        for t in range(NCHG):
            pltpu.make_async_copy(idx_u_hbm.at[pl.ds(gbase, GCH)],
                                  iu[t], si).wait()
            pltpu.make_async_copy(idx_c_hbm.at[pl.ds(gbase, GCH)],
                                  ic[t], si).wait()
        pltpu.make_async_copy(ones16_hbm, ones_v, si).wait()
        for t in range(NCHG):
            pltpu.async_copy(emb_u_hbm.at[iu[t]], bu[t], sg)
            pltpu.async_copy(emb_c_hbm.at[ic[t]], bc[t], sg)

        def start_didx(j, s):
            pltpu.async_copy(dst_hbm.at[pl.ds(ebase + j * ECH, ECH)],
                             didx[s], sd[s])

        def wait_didx(s):
            pltpu.make_async_copy(dst_hbm.at[pl.ds(ebase, ECH)],
                                  didx[s], sd[s]).wait()

        for t in range(4):
            start_didx(t, t)
        plsc.subcore_barrier()

        def deg_body(i, carry):
            for kk in range(8):
                j = 8 * i + kk
                s = kk
                wait_didx(s)
                pltpu.sync_copy(ones_v, deg_sh.at[didx[s]], add=True)
                start_didx(j + 4, (kk + 4) % 8)
            return carry

        lax.fori_loop(0, NCHE // 8, deg_body, 0)
        for kk in range(4):
            wait_didx(kk)

        for t in range(NCHG):
            pltpu.make_async_copy(emb_u_hbm.at[iu[t]], bu[t], sg).wait()
            pltpu.make_async_copy(emb_c_hbm.at[ic[t]], bc[t], sg).wait()
        for t in range(NCHG):
            pltpu.async_copy(bu[t], eu_hbm.at[pl.ds(gbase + t * GCH, GCH)],
                             swb)
            pltpu.async_copy(bc[t], ec_hbm.at[pl.ds(gbase + t * GCH, GCH)],
                             swb)
        for t in range(NCHG):
            pltpu.make_async_copy(bu[t], eu_hbm.at[pl.ds(gbase, GCH)],
                                  swb).wait()
            pltpu.make_async_copy(bc[t], ec_hbm.at[pl.ds(gbase, GCH)],
                                  swb).wait()

        plsc.subcore_barrier()
        pltpu.sync_copy(deg_sh.at[pl.ds(sid * ROWS_PER_SUB, ROWS_PER_SUB)],
                        deg_hbm.at[cid, pl.ds(sid * ROWS_PER_SUB, ROWS_PER_SUB)])

    return k(idx_u, idx_c, dst_pad, emb_user, emb_cat, zeros16, ones16)


def _sc_spmm64(y, src_pad, dst_pad, zeros, rows, nche, epw):
    rps = rows // NSUB

    @functools.partial(
        pl.kernel,
        mesh=_sc_mesh(),
        compiler_params=pltpu.CompilerParams(use_tc_tiling_on_sc=False),
        out_type=jax.ShapeDtypeStruct((2, rows, 64), jnp.float32),
        scratch_types=(
            [pltpu.VMEM((ECH,), jnp.int32) for _ in range(2 * NBUF)]
            + [pltpu.VMEM((ECH, 64), jnp.float32) for _ in range(NBUF)]
            + [pltpu.VMEM_SHARED((rows, 64), jnp.float32)]
            + [pltpu.SemaphoreType.DMA for _ in range(3 * NBUF)]
        ),
    )
    def k(y_hbm, src_hbm, dst_hbm, zeros_hbm, z_hbm, *scr):
        sidx = scr[0:NBUF]
        didx = scr[NBUF:2 * NBUF]
        buf = scr[2 * NBUF:3 * NBUF]
        z_sh = scr[3 * NBUF]
        sems = scr[3 * NBUF + 1:]
        ss = sems[0:NBUF]
        sd = sems[NBUF:2 * NBUF]
        sg = sems[2 * NBUF:3 * NBUF]

        cid = lax.axis_index("c")
        sid = lax.axis_index("s")
        wid = cid * NSUB + sid
        ebase = wid * epw

        def start_sidx(j, s):
            pltpu.async_copy(src_hbm.at[pl.ds(ebase + j * ECH, ECH)],
                             sidx[s], ss[s])

        def start_didx(j, s):
            pltpu.async_copy(dst_hbm.at[pl.ds(ebase + j * ECH, ECH)],
                             didx[s], sd[s])

        def wait_sidx(s):
            pltpu.make_async_copy(src_hbm.at[pl.ds(ebase, ECH)],
                                  sidx[s], ss[s]).wait()

        def wait_didx(s):
            pltpu.make_async_copy(dst_hbm.at[pl.ds(ebase, ECH)],
                                  didx[s], sd[s]).wait()

        def start_gather(s):
            pltpu.async_copy(y_hbm.at[sidx[s]], buf[s], sg[s])

        def wait_gather(s):
            pltpu.make_async_copy(y_hbm.at[sidx[s]], buf[s], sg[s]).wait()

        for t in range(4):
            start_sidx(t, t % NBUF)
        for t in range(3):
            start_didx(t, t % NBUF)
        pltpu.sync_copy(zeros_hbm.at[pl.ds(sid * rps, rps)],
                        z_sh.at[pl.ds(sid * rps, rps)])
        plsc.subcore_barrier()
        for t in range(3):
            wait_sidx(t % NBUF)
            start_gather(t % NBUF)

        def body(i, carry):
            for kk in range(20):
                j = 20 * i + kk
                b = kk % NBUF
                b3 = (kk + 3) % NBUF
                b4 = (kk + 4) % NBUF
                wait_gather(b)
                start_sidx(j + 4, b4)
                wait_didx(b)
                pltpu.sync_copy(buf[b], z_sh.at[didx[b]], add=True)
                start_didx(j + 3, b3)
                wait_sidx(b3)
                start_gather(b3)
            return carry

        lax.fori_loop(0, nche // 20, body, 0)

        for c in (nche, nche + 1, nche + 2):
            wait_gather(c % NBUF)
        wait_sidx((nche + 3) % NBUF)
        for c in (nche, nche + 1, nche + 2):
            wait_didx(c % NBUF)

        plsc.subcore_barrier()
        pltpu.sync_copy(z_sh.at[pl.ds(sid * rps, rps)],
                        z_hbm.at[cid, pl.ds(sid * rps, rps)])

    return k(y, src_pad, dst_pad, zeros)


_RB = 1280
_NRB = NP // _RB


def _tc_front(eu, ec, kn, fl, lm, degA, degB, emb_known, W_user, b_user,
              emb_mask, W_mask, b_mask, W_cat, b_cat, W0):
    def body(eu_r, ec_r, kn_r, fl_r, lm_r, dA_r, dB_r, ek_r, Wu_r, bu_r,
             em_r, Wm_r, bm_r, Wc_r, bc_r, W0_r, y0_r):
        eu_b = eu_r[...]
        kn_b = kn_r[...]
        ksel = jnp.where(kn_b == 0, ek_r[0:1, :], ek_r[1:2, :])
        uf = jnp.maximum(eu_b + ksel, 0.0) @ Wu_r[...] + bu_r[...]
        cf = jnp.maximum(ec_r[...], 0.0) @ Wc_r[...] + bc_r[...]
        mrows = jax.nn.sigmoid(jnp.maximum(em_r[...], 0.0) @ Wm_r[...] + bm_r[...])
        mf = jnp.where(lm_r[...] == 0, mrows[0:1, :], mrows[1:2, :])
        x = jnp.where(fl_r[...] == 0, uf, cf) * mf
        deg = dA_r[...][:, 0:1] + dB_r[...][:, 0:1] + 1.0
        dinv = lax.rsqrt(deg)
        y0_r[...] = dinv * (x @ W0_r[...])

    full = lambda shape: pl.BlockSpec(shape, lambda i: (0, 0))
    return pl.pallas_call(
        body,
        grid=(_NRB,),
        in_specs=[
            pl.BlockSpec((_RB, 64), lambda i: (i, 0)),
            pl.BlockSpec((_RB, 32), lambda i: (i, 0)),
            pl.BlockSpec((_RB, 1), lambda i: (i, 0)),
            pl.BlockSpec((_RB, 1), lambda i: (i, 0)),
            pl.BlockSpec((_RB, 1), lambda i: (i, 0)),
            pl.BlockSpec((_RB, 16), lambda i: (i, 0)),
            pl.BlockSpec((_RB, 16), lambda i: (i, 0)),
            full((2, 64)),
            full((64, 128)),
            full((1, 128)),
            full((2, 64)),
            full((64, 128)),
            full((1, 128)),
            full((32, 128)),
            full((1, 128)),
            full((128, 128)),
        ],
        out_specs=pl.BlockSpec((_RB, 128), lambda i: (i, 0)),
        out_shape=jax.ShapeDtypeStruct((NP, 128), jnp.float32),
    )(eu, ec, kn, fl, lm, degA, degB, emb_known, W_user, b_user,
      emb_mask, W_mask, b_mask, W_cat, b_cat, W0)


def _tc_mid(z0a, z0b, y0, degA, degB, b0, W2):
    def body(za_r, zb_r, y0_r, dA_r, dB_r, b0_r, W2_r, y2_r):
        deg = dA_r[...][:, 0:1] + dB_r[...][:, 0:1] + 1.0
        dinv = lax.rsqrt(deg)
        out0 = dinv * (za_r[...] + zb_r[...] + y0_r[...]) + b0_r[...]
        x1 = jnp.maximum(out0, 0.0)
        y2_r[...] = dinv * (x1 @ W2_r[...])

    full = lambda shape: pl.BlockSpec(shape, lambda i: (0, 0))
    return pl.pallas_call(
        body,
        grid=(_NRB,),
        in_specs=[
            pl.BlockSpec((_RB, 128), lambda i: (i, 0)),
            pl.BlockSpec((_RB, 128), lambda i: (i, 0)),
            pl.BlockSpec((_RB, 128), lambda i: (i, 0)),
            pl.BlockSpec((_RB, 16), lambda i: (i, 0)),
            pl.BlockSpec((_RB, 16), lambda i: (i, 0)),
            full((1, 128)),
            full((128, 64)),
        ],
        out_specs=pl.BlockSpec((_RB, 64), lambda i: (i, 0)),
        out_shape=jax.ShapeDtypeStruct((NP, 64), jnp.float32),
    )(z0a, z0b, y0, degA, degB, b0, W2)


def _tc_tail(z2a, z2b, y2, degA, degB, b2):
    def body(za_r, zb_r, y2_r, dA_r, dB_r, b2_r, out_r):
        deg = dA_r[...][:, 0:1] + dB_r[...][:, 0:1] + 1.0
        dinv = lax.rsqrt(deg)
        out_r[...] = dinv * (za_r[...] + zb_r[...] + y2_r[...]) + b2_r[...]

    full = lambda shape: pl.BlockSpec(shape, lambda i: (0, 0))
    return pl.pallas_call(
        body,
        grid=(_NRB,),
        in_specs=[
            pl.BlockSpec((_RB, 64), lambda i: (i, 0)),
            pl.BlockSpec((_RB, 64), lambda i: (i, 0)),
            pl.BlockSpec((_RB, 64), lambda i: (i, 0)),
            pl.BlockSpec((_RB, 16), lambda i: (i, 0)),
            pl.BlockSpec((_RB, 16), lambda i: (i, 0)),
            full((1, 64)),
        ],
        out_specs=pl.BlockSpec((_RB, 64), lambda i: (i, 0)),
        out_shape=jax.ShapeDtypeStruct((NP, 64), jnp.float32),
    )(z2a, z2b, y2, degA, degB, b2)


def kernel(edges, features, label_masks, emb_user, emb_known, W_user, b_user,
           emb_mask, W_mask, b_mask, emb_cat, W_cat, b_cat,
           W0, b0, W1, b1, W2, b2):
    idx = features[:, 0]
    known = features[:, 1]
    flag = features[:, 2]

    pad_n = NP - N
    idx_u = jnp.concatenate([idx, jnp.zeros((pad_n,), jnp.int32)])
    idx_c = jnp.concatenate([jnp.clip(idx, 0, CAT - 1),
                             jnp.zeros((pad_n,), jnp.int32)])
    kn = jnp.concatenate([known, jnp.zeros((pad_n,), jnp.int32)]).reshape(NP, 1)
    fl = jnp.concatenate([flag, jnp.zeros((pad_n,), jnp.int32)]).reshape(NP, 1)
    lm = jnp.concatenate([label_masks,
                          jnp.zeros((pad_n,), jnp.int32)]).reshape(NP, 1)

    real_per_w = E // NW
    src_w = jnp.pad(edges[0].reshape(NW, real_per_w),
                    ((0, 0), (0, EPW - real_per_w)), constant_values=N)
    dst_w = jnp.pad(edges[1].reshape(NW, real_per_w),
                    ((0, 0), (0, EPW - real_per_w)), constant_values=N)
    src_pad = src_w.reshape(-1)
    dst_pad = dst_w.reshape(-1)
    two = jnp.arange(2, dtype=jnp.int32)
    src2 = (2 * src_w[:, :, None] + two).reshape(-1)
    dst2 = (2 * dst_w[:, :, None] + two).reshape(-1)

    zeros2x64 = jnp.zeros((2 * NP, 64), jnp.float32)
    zeros64 = jnp.zeros((NP, 64), jnp.float32)
    zeros16 = jnp.zeros((NP, 16), jnp.float32)
    ones16 = jnp.ones((ECH, 16), jnp.float32)

    eu, ec, degp = _sc_front(idx_u, idx_c, dst_pad, emb_user, emb_cat,
                             zeros16, ones16)
    degA, degB = degp[0], degp[1]

    y0 = _tc_front(eu, ec, kn, fl, lm, degA, degB, emb_known, W_user,
                   b_user.reshape(1, -1), emb_mask, W_mask,
                   b_mask.reshape(1, -1), W_cat, b_cat.reshape(1, -1), W0)

    z0 = _sc_spmm64(y0.reshape(2 * NP, 64), src2, dst2, zeros2x64,
                    2 * NP, 2 * NCHE, 2 * EPW)
    z0 = z0.reshape(2, NP, 128)
    y2 = _tc_mid(z0[0], z0[1], y0, degA, degB, b0.reshape(1, -1), W2)
    z2 = _sc_spmm64(y2, src_pad, dst_pad, zeros64, NP, NCHE, EPW)
    out = _tc_tail(z2[0], z2[1], y2, degA, degB, b2.reshape(1, -1))
    return out[:N]


# speedup vs baseline: 1.6428x; 1.6428x over previous
import functools

import jax
import jax.numpy as jnp
from jax import lax
from jax.experimental import pallas as pl
from jax.experimental.pallas import tpu as pltpu
from jax.experimental.pallas import tpu_sc as plsc

N = 10000
E = 320000
CAT = 1000

NW = 32
NSUB = 16
NP = 10240
ECH = 128
NCHE = 80
EPW = NCHE * ECH
GCH = 64
NCHG = 5
ROWS_PER_SUB = NP // NSUB

CH0 = 104
CH1 = 56


def _sc_mesh():
    return plsc.VectorSubcoreMesh(core_axis_name="c", subcore_axis_name="s")


def _sc_front(idx_u, idx_c, dst_pad, emb_user, emb_cat, zeros16, ones16):
    @functools.partial(
        pl.kernel,
        mesh=_sc_mesh(),
        compiler_params=pltpu.CompilerParams(use_tc_tiling_on_sc=False),
        out_type=[
            jax.ShapeDtypeStruct((NP, 64), jnp.float32),
            jax.ShapeDtypeStruct((NP, 32), jnp.float32),
            jax.ShapeDtypeStruct((2, NP, 16), jnp.float32),
        ],
        scratch_types=[
            pltpu.VMEM((GCH,), jnp.int32),
            pltpu.VMEM((GCH, 64), jnp.float32),
            pltpu.VMEM((GCH, 32), jnp.float32),
            pltpu.VMEM((ECH,), jnp.int32),
            pltpu.VMEM((ECH, 16), jnp.float32),
            pltpu.VMEM_SHARED((NP, 16), jnp.float32),
            pltpu.SemaphoreType.DMA,
        ],
    )
    def k(idx_u_hbm, idx_c_hbm, dst_hbm, emb_u_hbm, emb_c_hbm,
          zeros16_hbm, ones16_hbm,
          eu_hbm, ec_hbm, deg_hbm,
          idxg, bufu, bufc, idxe, ones_v, deg_sh, sem):
        cid = lax.axis_index("c")
        sid = lax.axis_index("s")
        wid = cid * NSUB + sid
        gbase = wid * (NCHG * GCH)
        ebase = wid * EPW

        pltpu.sync_copy(zeros16_hbm.at[pl.ds(sid * ROWS_PER_SUB, ROWS_PER_SUB)],
                        deg_sh.at[pl.ds(sid * ROWS_PER_SUB, ROWS_PER_SUB)])
        pltpu.sync_copy(ones16_hbm, ones_v)
        plsc.subcore_barrier()

        def deg_body(j, carry):
            pltpu.sync_copy(dst_hbm.at[pl.ds(ebase + j * ECH, ECH)], idxe)
            pltpu.sync_copy(ones_v, deg_sh.at[idxe], add=True)
            return carry

        lax.fori_loop(0, NCHE, deg_body, 0)

        def g_body(j, carry):
            base = gbase + j * GCH
            pltpu.sync_copy(idx_u_hbm.at[pl.ds(base, GCH)], idxg)
            pltpu.async_copy(emb_u_hbm.at[idxg], bufu, sem).wait()
            pltpu.sync_copy(bufu, eu_hbm.at[pl.ds(base, GCH)])
            pltpu.sync_copy(idx_c_hbm.at[pl.ds(base, GCH)], idxg)
            pltpu.async_copy(emb_c_hbm.at[idxg], bufc, sem).wait()
            pltpu.sync_copy(bufc, ec_hbm.at[pl.ds(base, GCH)])
            return carry

        lax.fori_loop(0, NCHG, g_body, 0)

        plsc.subcore_barrier()
        pltpu.sync_copy(deg_sh.at[pl.ds(sid * ROWS_PER_SUB, ROWS_PER_SUB)],
                        deg_hbm.at[cid, pl.ds(sid * ROWS_PER_SUB, ROWS_PER_SUB)])

    return k(idx_u, idx_c, dst_pad, emb_user, emb_cat, zeros16, ones16)


def _sc_spmm(y, src_pad, dst_pad, zeros, D):
    @functools.partial(
        pl.kernel,
        mesh=_sc_mesh(),
        compiler_params=pltpu.CompilerParams(use_tc_tiling_on_sc=False),
        out_type=jax.ShapeDtypeStruct((2, NP, D), jnp.float32),
        scratch_types=[
            pltpu.VMEM((ECH,), jnp.int32),
            pltpu.VMEM((ECH,), jnp.int32),
            pltpu.VMEM((ECH,), jnp.int32),
            pltpu.VMEM((ECH,), jnp.int32),
            pltpu.VMEM((ECH, D), jnp.float32),
            pltpu.VMEM((ECH, D), jnp.float32),
            pltpu.VMEM_SHARED((NP, D), jnp.float32),
            pltpu.SemaphoreType.DMA,
            pltpu.SemaphoreType.DMA,
            pltpu.SemaphoreType.DMA,
            pltpu.SemaphoreType.DMA,
            pltpu.SemaphoreType.DMA,
            pltpu.SemaphoreType.DMA,
        ],
    )
    def k(y_hbm, src_hbm, dst_hbm, zeros_hbm, z_hbm,
          sidx0, sidx1, didx0, didx1, buf0, buf1, z_sh,
          ss0, ss1, sd0, sd1, sg0, sg1):
        cid = lax.axis_index("c")
        sid = lax.axis_index("s")
        nche = jnp.where(cid == 0, CH0, CH1)
        ebase = jnp.where(cid == 0, sid * CH0, NSUB * CH0 + sid * CH1) * ECH
        sidx = (sidx0, sidx1)
        didx = (didx0, didx1)
        buf = (buf0, buf1)
        ss = (ss0, ss1)
        sd = (sd0, sd1)
        sg = (sg0, sg1)

        def start_idx(j, b):
            pltpu.async_copy(src_hbm.at[pl.ds(ebase + j * ECH, ECH)],
                             sidx[b], ss[b])
            pltpu.async_copy(dst_hbm.at[pl.ds(ebase + j * ECH, ECH)],
                             didx[b], sd[b])

        def wait_sidx(b):
            pltpu.make_async_copy(src_hbm.at[pl.ds(ebase, ECH)],
                                  sidx[b], ss[b]).wait()

        def wait_didx(b):
            pltpu.make_async_copy(dst_hbm.at[pl.ds(ebase, ECH)],
                                  didx[b], sd[b]).wait()

        start_idx(0, 0)
        start_idx(1, 1)
        pltpu.sync_copy(zeros_hbm.at[pl.ds(sid * ROWS_PER_SUB, ROWS_PER_SUB)],
                        z_sh.at[pl.ds(sid * ROWS_PER_SUB, ROWS_PER_SUB)])
        plsc.subcore_barrier()
        wait_sidx(0)
        pltpu.async_copy(y_hbm.at[sidx[0]], buf[0], sg[0])

        def body(i, carry):
            for b in range(2):
                j = 2 * i + b
                nb = 1 - b
                pltpu.make_async_copy(y_hbm.at[sidx[b]], buf[b], sg[b]).wait()

                @pl.when(j + 1 < nche)
                def _():
                    wait_sidx(nb)
                    pltpu.async_copy(y_hbm.at[sidx[nb]], buf[nb], sg[nb])

                wait_didx(b)
                pltpu.sync_copy(buf[b], z_sh.at[didx[b]], add=True)

                @pl.when(j + 2 < nche)
                def _():
                    start_idx(j + 2, b)
            return carry

        lax.fori_loop(0, nche // 2, body, 0)

        plsc.subcore_barrier()
        pltpu.sync_copy(z_sh.at[pl.ds(sid * ROWS_PER_SUB, ROWS_PER_SUB)],
                        z_hbm.at[cid, pl.ds(sid * ROWS_PER_SUB, ROWS_PER_SUB)])

    return k(y, src_pad, dst_pad, zeros)


_RB = 1280
_NRB = NP // _RB


def _tc_front(eu, ec, kn, fl, lm, degA, degB, emb_known, W_user, b_user,
              emb_mask, W_mask, b_mask, W_cat, b_cat, W0):
    def body(eu_r, ec_r, kn_r, fl_r, lm_r, dA_r, dB_r, ek_r, Wu_r, bu_r,
             em_r, Wm_r, bm_r, Wc_r, bc_r, W0_r, y0_r):
        eu_b = eu_r[...]
        kn_b = kn_r[...]
        ksel = jnp.where(kn_b == 0, ek_r[0:1, :], ek_r[1:2, :])
        uf = jnp.maximum(eu_b + ksel, 0.0) @ Wu_r[...] + bu_r[...]
        cf = jnp.maximum(ec_r[...], 0.0) @ Wc_r[...] + bc_r[...]
        mrows = jax.nn.sigmoid(jnp.maximum(em_r[...], 0.0) @ Wm_r[...] + bm_r[...])
        mf = jnp.where(lm_r[...] == 0, mrows[0:1, :], mrows[1:2, :])
        x = jnp.where(fl_r[...] == 0, uf, cf) * mf
        deg = dA_r[...][:, 0:1] + dB_r[...][:, 0:1] + 1.0
        dinv = lax.rsqrt(deg)
        y0_r[...] = dinv * (x @ W0_r[...])

    full = lambda shape: pl.BlockSpec(shape, lambda i: (0, 0))
    return pl.pallas_call(
        body,
        grid=(_NRB,),
        in_specs=[
            pl.BlockSpec((_RB, 64), lambda i: (i, 0)),
            pl.BlockSpec((_RB, 32), lambda i: (i, 0)),
            pl.BlockSpec((_RB, 1), lambda i: (i, 0)),
            pl.BlockSpec((_RB, 1), lambda i: (i, 0)),
            pl.BlockSpec((_RB, 1), lambda i: (i, 0)),
            pl.BlockSpec((_RB, 16), lambda i: (i, 0)),
            pl.BlockSpec((_RB, 16), lambda i: (i, 0)),
            full((2, 64)),
            full((64, 128)),
            full((1, 128)),
            full((2, 64)),
            full((64, 128)),
            full((1, 128)),
            full((32, 128)),
            full((1, 128)),
            full((128, 128)),
        ],
        out_specs=pl.BlockSpec((_RB, 128), lambda i: (i, 0)),
        out_shape=jax.ShapeDtypeStruct((NP, 128), jnp.float32),
    )(eu, ec, kn, fl, lm, degA, degB, emb_known, W_user, b_user,
      emb_mask, W_mask, b_mask, W_cat, b_cat, W0)


def _tc_mid(z0a, z0b, y0, degA, degB, b0, W2):
    def body(za_r, zb_r, y0_r, dA_r, dB_r, b0_r, W2_r, y2_r):
        deg = dA_r[...][:, 0:1] + dB_r[...][:, 0:1] + 1.0
        dinv = lax.rsqrt(deg)
        out0 = dinv * (za_r[...] + zb_r[...] + y0_r[...]) + b0_r[...]
        x1 = jnp.maximum(out0, 0.0)
        y2_r[...] = dinv * (x1 @ W2_r[...])

    full = lambda shape: pl.BlockSpec(shape, lambda i: (0, 0))
    return pl.pallas_call(
        body,
        grid=(_NRB,),
        in_specs=[
            pl.BlockSpec((_RB, 128), lambda i: (i, 0)),
            pl.BlockSpec((_RB, 128), lambda i: (i, 0)),
            pl.BlockSpec((_RB, 128), lambda i: (i, 0)),
            pl.BlockSpec((_RB, 16), lambda i: (i, 0)),
            pl.BlockSpec((_RB, 16), lambda i: (i, 0)),
            full((1, 128)),
            full((128, 64)),
        ],
        out_specs=pl.BlockSpec((_RB, 64), lambda i: (i, 0)),
        out_shape=jax.ShapeDtypeStruct((NP, 64), jnp.float32),
    )(z0a, z0b, y0, degA, degB, b0, W2)


def _tc_tail(z2a, z2b, y2, degA, degB, b2):
    def body(za_r, zb_r, y2_r, dA_r, dB_r, b2_r, out_r):
        deg = dA_r[...][:, 0:1] + dB_r[...][:, 0:1] + 1.0
        dinv = lax.rsqrt(deg)
        out_r[...] = dinv * (za_r[...] + zb_r[...] + y2_r[...]) + b2_r[...]

    full = lambda shape: pl.BlockSpec(shape, lambda i: (0, 0))
    return pl.pallas_call(
        body,
        grid=(_NRB,),
        in_specs=[
            pl.BlockSpec((_RB, 64), lambda i: (i, 0)),
            pl.BlockSpec((_RB, 64), lambda i: (i, 0)),
            pl.BlockSpec((_RB, 64), lambda i: (i, 0)),
            pl.BlockSpec((_RB, 16), lambda i: (i, 0)),
            pl.BlockSpec((_RB, 16), lambda i: (i, 0)),
            full((1, 64)),
        ],
        out_specs=pl.BlockSpec((_RB, 64), lambda i: (i, 0)),
        out_shape=jax.ShapeDtypeStruct((NP, 64), jnp.float32),
    )(z2a, z2b, y2, degA, degB, b2)


def kernel(edges, features, label_masks, emb_user, emb_known, W_user, b_user,
           emb_mask, W_mask, b_mask, emb_cat, W_cat, b_cat,
           W0, b0, W1, b1, W2, b2):
    idx = features[:, 0]
    known = features[:, 1]
    flag = features[:, 2]

    pad_n = NP - N
    idx_u = jnp.concatenate([idx, jnp.zeros((pad_n,), jnp.int32)])
    idx_c = jnp.concatenate([jnp.clip(idx, 0, CAT - 1),
                             jnp.zeros((pad_n,), jnp.int32)])
    kn = jnp.concatenate([known, jnp.zeros((pad_n,), jnp.int32)]).reshape(NP, 1)
    fl = jnp.concatenate([flag, jnp.zeros((pad_n,), jnp.int32)]).reshape(NP, 1)
    lm = jnp.concatenate([label_masks,
                          jnp.zeros((pad_n,), jnp.int32)]).reshape(NP, 1)

    pad_e = NW * EPW - E
    pad_idx = jnp.full((pad_e,), N, jnp.int32)
    src_pad = jnp.concatenate([edges[0], pad_idx])
    dst_pad = jnp.concatenate([edges[1], pad_idx])

    zeros128 = jnp.zeros((NP, 128), jnp.float32)
    zeros64 = jnp.zeros((NP, 64), jnp.float32)
    zeros16 = jnp.zeros((NP, 16), jnp.float32)
    ones16 = jnp.ones((ECH, 16), jnp.float32)

    eu, ec, degp = _sc_front(idx_u, idx_c, dst_pad, emb_user, emb_cat,
                             zeros16, ones16)
    degA, degB = degp[0], degp[1]

    y0 = _tc_front(eu, ec, kn, fl, lm, degA, degB, emb_known, W_user,
                   b_user.reshape(1, -1), emb_mask, W_mask,
                   b_mask.reshape(1, -1), W_cat, b_cat.reshape(1, -1), W0)

    z0 = _sc_spmm(y0, src_pad, dst_pad, zeros128, 128)
    y2 = _tc_mid(z0[0], z0[1], y0, degA, degB, b0.reshape(1, -1), W2)
    z2 = _sc_spmm(y2, src_pad, dst_pad, zeros64, 64)
    out = _tc_tail(z2[0], z2[1], y2, degA, degB, b2.reshape(1, -1))
    return out[:N]


# speedup vs baseline: 1.7426x; 1.0608x over previous
import functools

import jax
import jax.numpy as jnp
from jax import lax
from jax.experimental import pallas as pl
from jax.experimental.pallas import tpu as pltpu
from jax.experimental.pallas import tpu_sc as plsc

N = 10000
E = 320000
CAT = 1000

NW = 32
NSUB = 16
NP = 10240
ECH = 128
NCHE = 80
EPW = NCHE * ECH
GCH = 64
NCHG = 5
ROWS_PER_SUB = NP // NSUB

CH0 = 128
CH1 = 32


def _sc_mesh():
    return plsc.VectorSubcoreMesh(core_axis_name="c", subcore_axis_name="s")


def _sc_front(idx_u, idx_c, dst_pad, emb_user, emb_cat, zeros16, ones16):
    @functools.partial(
        pl.kernel,
        mesh=_sc_mesh(),
        compiler_params=pltpu.CompilerParams(use_tc_tiling_on_sc=False),
        out_type=[
            jax.ShapeDtypeStruct((NP, 64), jnp.float32),
            jax.ShapeDtypeStruct((NP, 32), jnp.float32),
            jax.ShapeDtypeStruct((2, NP, 16), jnp.float32),
        ],
        scratch_types=[
            pltpu.VMEM((GCH,), jnp.int32),
            pltpu.VMEM((GCH, 64), jnp.float32),
            pltpu.VMEM((GCH, 32), jnp.float32),
            pltpu.VMEM((ECH,), jnp.int32),
            pltpu.VMEM((ECH,), jnp.int32),
            pltpu.VMEM((ECH, 16), jnp.float32),
            pltpu.VMEM_SHARED((NP, 16), jnp.float32),
            pltpu.SemaphoreType.DMA,
            pltpu.SemaphoreType.DMA,
            pltpu.SemaphoreType.DMA,
        ],
    )
    def k(idx_u_hbm, idx_c_hbm, dst_hbm, emb_u_hbm, emb_c_hbm,
          zeros16_hbm, ones16_hbm,
          eu_hbm, ec_hbm, deg_hbm,
          idxg, bufu, bufc, idxe0, idxe1, ones_v, deg_sh, sem, sd0, sd1):
        cid = lax.axis_index("c")
        sid = lax.axis_index("s")
        wid = cid * NSUB + sid
        gbase = wid * (NCHG * GCH)
        ebase = wid * EPW
        idxe = (idxe0, idxe1)
        sd = (sd0, sd1)

        def start_didx(j, b):
            pltpu.async_copy(dst_hbm.at[pl.ds(ebase + j * ECH, ECH)],
                             idxe[b], sd[b])

        def wait_didx(b):
            pltpu.make_async_copy(dst_hbm.at[pl.ds(ebase, ECH)],
                                  idxe[b], sd[b]).wait()

        start_didx(0, 0)
        start_didx(1, 1)
        pltpu.sync_copy(zeros16_hbm.at[pl.ds(sid * ROWS_PER_SUB, ROWS_PER_SUB)],
                        deg_sh.at[pl.ds(sid * ROWS_PER_SUB, ROWS_PER_SUB)])
        pltpu.sync_copy(ones16_hbm, ones_v)
        plsc.subcore_barrier()

        def deg_body(i, carry):
            for b in range(2):
                j = 2 * i + b
                wait_didx(b)
                pltpu.sync_copy(ones_v, deg_sh.at[idxe[b]], add=True)

                @pl.when(j + 2 < NCHE)
                def _():
                    start_didx(j + 2, b)
            return carry

        lax.fori_loop(0, NCHE // 2, deg_body, 0)

        def g_body(j, carry):
            base = gbase + j * GCH
            pltpu.sync_copy(idx_u_hbm.at[pl.ds(base, GCH)], idxg)
            pltpu.async_copy(emb_u_hbm.at[idxg], bufu, sem).wait()
            pltpu.sync_copy(bufu, eu_hbm.at[pl.ds(base, GCH)])
            pltpu.sync_copy(idx_c_hbm.at[pl.ds(base, GCH)], idxg)
            pltpu.async_copy(emb_c_hbm.at[idxg], bufc, sem).wait()
            pltpu.sync_copy(bufc, ec_hbm.at[pl.ds(base, GCH)])
            return carry

        lax.fori_loop(0, NCHG, g_body, 0)

        plsc.subcore_barrier()
        pltpu.sync_copy(deg_sh.at[pl.ds(sid * ROWS_PER_SUB, ROWS_PER_SUB)],
                        deg_hbm.at[cid, pl.ds(sid * ROWS_PER_SUB, ROWS_PER_SUB)])

    return k(idx_u, idx_c, dst_pad, emb_user, emb_cat, zeros16, ones16)


def _sc_spmm(y, src_pad, dst_pad, zeros, D):
    @functools.partial(
        pl.kernel,
        mesh=_sc_mesh(),
        compiler_params=pltpu.CompilerParams(use_tc_tiling_on_sc=False),
        out_type=jax.ShapeDtypeStruct((2, NP, D), jnp.float32),
        scratch_types=[
            pltpu.VMEM((ECH,), jnp.int32),
            pltpu.VMEM((ECH,), jnp.int32),
            pltpu.VMEM((ECH,), jnp.int32),
            pltpu.VMEM((ECH,), jnp.int32),
            pltpu.VMEM((ECH, D), jnp.float32),
            pltpu.VMEM((ECH, D), jnp.float32),
            pltpu.VMEM_SHARED((NP, D), jnp.float32),
            pltpu.SemaphoreType.DMA,
            pltpu.SemaphoreType.DMA,
            pltpu.SemaphoreType.DMA,
            pltpu.SemaphoreType.DMA,
            pltpu.SemaphoreType.DMA,
            pltpu.SemaphoreType.DMA,
        ],
    )
    def k(y_hbm, src_hbm, dst_hbm, zeros_hbm, z_hbm,
          sidx0, sidx1, didx0, didx1, buf0, buf1, z_sh,
          ss0, ss1, sd0, sd1, sg0, sg1):
        cid = lax.axis_index("c")
        sid = lax.axis_index("s")
        nche = jnp.where(cid == 0, CH0, CH1)
        ebase = jnp.where(cid == 0, sid * CH0, NSUB * CH0 + sid * CH1) * ECH
        sidx = (sidx0, sidx1)
        didx = (didx0, didx1)
        buf = (buf0, buf1)
        ss = (ss0, ss1)
        sd = (sd0, sd1)
        sg = (sg0, sg1)

        def start_idx(j, b):
            pltpu.async_copy(src_hbm.at[pl.ds(ebase + j * ECH, ECH)],
                             sidx[b], ss[b])
            pltpu.async_copy(dst_hbm.at[pl.ds(ebase + j * ECH, ECH)],
                             didx[b], sd[b])

        def wait_sidx(b):
            pltpu.make_async_copy(src_hbm.at[pl.ds(ebase, ECH)],
                                  sidx[b], ss[b]).wait()

        def wait_didx(b):
            pltpu.make_async_copy(dst_hbm.at[pl.ds(ebase, ECH)],
                                  didx[b], sd[b]).wait()

        start_idx(0, 0)
        start_idx(1, 1)
        pltpu.sync_copy(zeros_hbm.at[pl.ds(sid * ROWS_PER_SUB, ROWS_PER_SUB)],
                        z_sh.at[pl.ds(sid * ROWS_PER_SUB, ROWS_PER_SUB)])
        plsc.subcore_barrier()
        wait_sidx(0)
        pltpu.async_copy(y_hbm.at[sidx[0]], buf[0], sg[0])

        def body(i, carry):
            for b in range(2):
                j = 2 * i + b
                nb = 1 - b
                pltpu.make_async_copy(y_hbm.at[sidx[b]], buf[b], sg[b]).wait()

                @pl.when(j + 1 < nche)
                def _():
                    wait_sidx(nb)
                    pltpu.async_copy(y_hbm.at[sidx[nb]], buf[nb], sg[nb])

                wait_didx(b)
                pltpu.sync_copy(buf[b], z_sh.at[didx[b]], add=True)

                @pl.when(j + 2 < nche)
                def _():
                    start_idx(j + 2, b)
            return carry

        lax.fori_loop(0, nche // 2, body, 0)

        plsc.subcore_barrier()
        pltpu.sync_copy(z_sh.at[pl.ds(sid * ROWS_PER_SUB, ROWS_PER_SUB)],
                        z_hbm.at[cid, pl.ds(sid * ROWS_PER_SUB, ROWS_PER_SUB)])

    return k(y, src_pad, dst_pad, zeros)


_RB = 1280
_NRB = NP // _RB


def _tc_front(eu, ec, kn, fl, lm, degA, degB, emb_known, W_user, b_user,
              emb_mask, W_mask, b_mask, W_cat, b_cat, W0):
    def body(eu_r, ec_r, kn_r, fl_r, lm_r, dA_r, dB_r, ek_r, Wu_r, bu_r,
             em_r, Wm_r, bm_r, Wc_r, bc_r, W0_r, y0_r):
        eu_b = eu_r[...]
        kn_b = kn_r[...]
        ksel = jnp.where(kn_b == 0, ek_r[0:1, :], ek_r[1:2, :])
        uf = jnp.maximum(eu_b + ksel, 0.0) @ Wu_r[...] + bu_r[...]
        cf = jnp.maximum(ec_r[...], 0.0) @ Wc_r[...] + bc_r[...]
        mrows = jax.nn.sigmoid(jnp.maximum(em_r[...], 0.0) @ Wm_r[...] + bm_r[...])
        mf = jnp.where(lm_r[...] == 0, mrows[0:1, :], mrows[1:2, :])
        x = jnp.where(fl_r[...] == 0, uf, cf) * mf
        deg = dA_r[...][:, 0:1] + dB_r[...][:, 0:1] + 1.0
        dinv = lax.rsqrt(deg)
        y0_r[...] = dinv * (x @ W0_r[...])

    full = lambda shape: pl.BlockSpec(shape, lambda i: (0, 0))
    return pl.pallas_call(
        body,
        grid=(_NRB,),
        in_specs=[
            pl.BlockSpec((_RB, 64), lambda i: (i, 0)),
            pl.BlockSpec((_RB, 32), lambda i: (i, 0)),
            pl.BlockSpec((_RB, 1), lambda i: (i, 0)),
            pl.BlockSpec((_RB, 1), lambda i: (i, 0)),
            pl.BlockSpec((_RB, 1), lambda i: (i, 0)),
            pl.BlockSpec((_RB, 16), lambda i: (i, 0)),
            pl.BlockSpec((_RB, 16), lambda i: (i, 0)),
            full((2, 64)),
            full((64, 128)),
            full((1, 128)),
            full((2, 64)),
            full((64, 128)),
            full((1, 128)),
            full((32, 128)),
            full((1, 128)),
            full((128, 128)),
        ],
        out_specs=pl.BlockSpec((_RB, 128), lambda i: (i, 0)),
        out_shape=jax.ShapeDtypeStruct((NP, 128), jnp.float32),
    )(eu, ec, kn, fl, lm, degA, degB, emb_known, W_user, b_user,
      emb_mask, W_mask, b_mask, W_cat, b_cat, W0)


def _tc_mid(z0a, z0b, y0, degA, degB, b0, W2):
    def body(za_r, zb_r, y0_r, dA_r, dB_r, b0_r, W2_r, y2_r):
        deg = dA_r[...][:, 0:1] + dB_r[...][:, 0:1] + 1.0
        dinv = lax.rsqrt(deg)
        out0 = dinv * (za_r[...] + zb_r[...] + y0_r[...]) + b0_r[...]
        x1 = jnp.maximum(out0, 0.0)
        y2_r[...] = dinv * (x1 @ W2_r[...])

    full = lambda shape: pl.BlockSpec(shape, lambda i: (0, 0))
    return pl.pallas_call(
        body,
        grid=(_NRB,),
        in_specs=[
            pl.BlockSpec((_RB, 128), lambda i: (i, 0)),
            pl.BlockSpec((_RB, 128), lambda i: (i, 0)),
            pl.BlockSpec((_RB, 128), lambda i: (i, 0)),
            pl.BlockSpec((_RB, 16), lambda i: (i, 0)),
            pl.BlockSpec((_RB, 16), lambda i: (i, 0)),
            full((1, 128)),
            full((128, 64)),
        ],
        out_specs=pl.BlockSpec((_RB, 64), lambda i: (i, 0)),
        out_shape=jax.ShapeDtypeStruct((NP, 64), jnp.float32),
    )(z0a, z0b, y0, degA, degB, b0, W2)


def _tc_tail(z2a, z2b, y2, degA, degB, b2):
    def body(za_r, zb_r, y2_r, dA_r, dB_r, b2_r, out_r):
        deg = dA_r[...][:, 0:1] + dB_r[...][:, 0:1] + 1.0
        dinv = lax.rsqrt(deg)
        out_r[...] = dinv * (za_r[...] + zb_r[...] + y2_r[...]) + b2_r[...]

    full = lambda shape: pl.BlockSpec(shape, lambda i: (0, 0))
    return pl.pallas_call(
        body,
        grid=(_NRB,),
        in_specs=[
            pl.BlockSpec((_RB, 64), lambda i: (i, 0)),
            pl.BlockSpec((_RB, 64), lambda i: (i, 0)),
            pl.BlockSpec((_RB, 64), lambda i: (i, 0)),
            pl.BlockSpec((_RB, 16), lambda i: (i, 0)),
            pl.BlockSpec((_RB, 16), lambda i: (i, 0)),
            full((1, 64)),
        ],
        out_specs=pl.BlockSpec((_RB, 64), lambda i: (i, 0)),
        out_shape=jax.ShapeDtypeStruct((NP, 64), jnp.float32),
    )(z2a, z2b, y2, degA, degB, b2)


def kernel(edges, features, label_masks, emb_user, emb_known, W_user, b_user,
           emb_mask, W_mask, b_mask, emb_cat, W_cat, b_cat,
           W0, b0, W1, b1, W2, b2):
    idx = features[:, 0]
    known = features[:, 1]
    flag = features[:, 2]

    pad_n = NP - N
    idx_u = jnp.concatenate([idx, jnp.zeros((pad_n,), jnp.int32)])
    idx_c = jnp.concatenate([jnp.clip(idx, 0, CAT - 1),
                             jnp.zeros((pad_n,), jnp.int32)])
    kn = jnp.concatenate([known, jnp.zeros((pad_n,), jnp.int32)]).reshape(NP, 1)
    fl = jnp.concatenate([flag, jnp.zeros((pad_n,), jnp.int32)]).reshape(NP, 1)
    lm = jnp.concatenate([label_masks,
                          jnp.zeros((pad_n,), jnp.int32)]).reshape(NP, 1)

    pad_e = NW * EPW - E
    pad_idx = jnp.full((pad_e,), N, jnp.int32)
    src_pad = jnp.concatenate([edges[0], pad_idx])
    dst_pad = jnp.concatenate([edges[1], pad_idx])

    zeros128 = jnp.zeros((NP, 128), jnp.float32)
    zeros64 = jnp.zeros((NP, 64), jnp.float32)
    zeros16 = jnp.zeros((NP, 16), jnp.float32)
    ones16 = jnp.ones((ECH, 16), jnp.float32)

    eu, ec, degp = _sc_front(idx_u, idx_c, dst_pad, emb_user, emb_cat,
                             zeros16, ones16)
    degA, degB = degp[0], degp[1]

    y0 = _tc_front(eu, ec, kn, fl, lm, degA, degB, emb_known, W_user,
                   b_user.reshape(1, -1), emb_mask, W_mask,
                   b_mask.reshape(1, -1), W_cat, b_cat.reshape(1, -1), W0)

    z0 = _sc_spmm(y0, src_pad, dst_pad, zeros128, 128)
    y2 = _tc_mid(z0[0], z0[1], y0, degA, degB, b0.reshape(1, -1), W2)
    z2 = _sc_spmm(y2, src_pad, dst_pad, zeros64, 64)
    out = _tc_tail(z2[0], z2[1], y2, degA, degB, b2.reshape(1, -1))
    return out[:N]
